# baseline (speedup 1.0000x reference)
import functools

import jax
import jax.numpy as jnp
from jax import lax
from jax.experimental import pallas as pl
from jax.experimental.pallas import tpu as pltpu
from jax.experimental.pallas import tpu_sc as plsc

N_EDGES = 320000
D = 128
NV = D // 16
NC, NS = 2, 16
NW = NC * NS
R = 2
NPR = 160
NRANGE = NW * R
NPAD = NRANGE * NPR
CHUNK = 128
FMAX = 3.4e38


def _sc_moments(x_pad, srcs, ews, nstarts):
    mesh = plsc.VectorSubcoreMesh(
        core_axis_name="c", subcore_axis_name="s",
        num_cores=NC, num_subcores=NS)
    out_type = [
        jax.ShapeDtypeStruct((NPAD * D,), jnp.float32),
        jax.ShapeDtypeStruct((NPAD * D,), jnp.float32),
        jax.ShapeDtypeStruct((NPAD * D,), jnp.float32),
        jax.ShapeDtypeStruct((NPAD * D,), jnp.float32),
        jax.ShapeDtypeStruct((NPAD * 16,), jnp.float32),
    ]
    scratch_types = [
        pltpu.VMEM((NPR * D,), jnp.float32),
        pltpu.VMEM((NPR * D,), jnp.float32),
        pltpu.VMEM((NPR * D,), jnp.float32),
        pltpu.VMEM((NPR * D,), jnp.float32),
        pltpu.VMEM((NPR * 16,), jnp.float32),
        pltpu.VMEM((2 * CHUNK, D), jnp.float32),
        pltpu.VMEM((2 * CHUNK,), jnp.int32),
        pltpu.VMEM((2 * CHUNK + 16,), jnp.float32),
        pltpu.VMEM((NPR + 16,), jnp.int32),
        pltpu.SemaphoreType.DMA,
        pltpu.SemaphoreType.DMA,
    ]

    @functools.partial(pl.kernel, out_type=out_type, mesh=mesh,
                       scratch_types=scratch_types)
    def body(x_hbm, srcs_hbm, ews_hbm, nstarts_hbm,
             o_sum, o_sq, o_mx, o_mn, o_deg,
             v_sum, v_sq, v_mx, v_mn, v_deg,
             v_rows, v_idx, v_ew, v_nst, sem0, sem1):
        wid = lax.axis_index("c") * NS + lax.axis_index("s")
        zeros = jnp.zeros((16,), jnp.float32)
        fmaxv = jnp.full((16,), FMAX, jnp.float32)
        sems = (sem0, sem1)

        def stage_issue(ck, h):
            cb = ck * CHUNK
            hb = h * CHUNK
            pltpu.sync_copy(srcs_hbm.at[pl.ds(cb, CHUNK)],
                            v_idx.at[pl.ds(hb, CHUNK)])
            pltpu.sync_copy(ews_hbm.at[pl.ds(cb, CHUNK)],
                            v_ew.at[pl.ds(hb, CHUNK)])
            pltpu.async_copy(x_hbm.at[v_idx.at[pl.ds(hb, CHUNK)]],
                             v_rows.at[pl.ds(hb, CHUNK)], sems[h])

        def wait_gather(h):
            hb = h * CHUNK
            pltpu.make_async_copy(
                x_hbm.at[v_idx.at[pl.ds(hb, CHUNK)]],
                v_rows.at[pl.ds(hb, CHUNK)], sems[h]).wait()

        def refill(c):
            c1 = c + 1
            odd = jnp.bitwise_and(c1, 1)

            @pl.when(odd == 0)
            def _():
                wait_gather(0)
                stage_issue(c1 + 1, 1)

            @pl.when(odd == 1)
            def _():
                wait_gather(1)
                stage_issue(c1 + 1, 0)

            return c1

        def edge_body(j, st):
            (c, eptr), (s_acc, q_acc, x_acc, n_acc) = st
            c, eptr = lax.cond(
                eptr == CHUNK,
                lambda: (refill(c), jnp.int32(0)),
                lambda: (c, eptr))
            off = jnp.bitwise_and(c, 1) * CHUNK + eptr
            w = v_ew[pl.ds(off, 16)][0]
            s_new, q_new, x_new, n_new = [], [], [], []
            for f in range(NV):
                row = v_rows[off, pl.ds(f * 16, 16)]
                m = row * w
                s_new.append(s_acc[f] + m)
                q_new.append(q_acc[f] + m * m)
                x_new.append(jnp.maximum(x_acc[f], m))
                n_new.append(jnp.minimum(n_acc[f], m))
            return ((c, eptr + 1),
                    (tuple(s_new), tuple(q_new),
                     tuple(x_new), tuple(n_new)))

        acc_fresh = (tuple(zeros for _ in range(NV)),
                     tuple(zeros for _ in range(NV)),
                     tuple(-fmaxv for _ in range(NV)),
                     tuple(fmaxv for _ in range(NV)))

        def node_body(i, st):
            nv = v_nst[pl.ds(i, 16)]
            ne = nv[1] - nv[0]
            st2 = lax.fori_loop(0, ne, edge_body, (st, acc_fresh))
            (c, eptr), (s_acc, q_acc, x_acc, n_acc) = st2
            lo = i * D
            for f in range(NV):
                sl = pl.ds(lo + f * 16, 16)
                v_sum[sl] = s_acc[f]
                v_sq[sl] = q_acc[f]
                v_mx[sl] = x_acc[f]
                v_mn[sl] = n_acc[f]
            v_deg[pl.ds(i * 16, 16)] = zeros + ne.astype(jnp.float32)
            return (c, eptr)

        for r in range(R):
            rid = wid * R + r
            base = rid * NPR
            pltpu.sync_copy(nstarts_hbm.at[pl.ds(base, NPR + 16)], v_nst)
            start = v_nst[pl.ds(0, 16)][0]
            c0 = start // CHUNK
            odd0 = jnp.bitwise_and(c0, 1)

            @pl.when(odd0 == 0)
            def _():
                stage_issue(c0, 0)
                stage_issue(c0 + 1, 1)
                wait_gather(0)

            @pl.when(odd0 == 1)
            def _():
                stage_issue(c0, 1)
                stage_issue(c0 + 1, 0)
                wait_gather(1)

            c, eptr = lax.fori_loop(
                0, NPR, node_body, (c0, start - c0 * CHUNK))

            odd1 = jnp.bitwise_and(c + 1, 1)

            @pl.when(odd1 == 0)
            def _():
                wait_gather(0)

            @pl.when(odd1 == 1)
            def _():
                wait_gather(1)

            pltpu.sync_copy(v_sum, o_sum.at[pl.ds(base * D, NPR * D)])
            pltpu.sync_copy(v_sq, o_sq.at[pl.ds(base * D, NPR * D)])
            pltpu.sync_copy(v_mx, o_mx.at[pl.ds(base * D, NPR * D)])
            pltpu.sync_copy(v_mn, o_mn.at[pl.ds(base * D, NPR * D)])
            pltpu.sync_copy(v_deg, o_deg.at[pl.ds(base * 16, NPR * 16)])

    return body(x_pad, srcs, ews, nstarts)


BLK = 1024


def _tc_layer_body(nreal, relu, deg_full_ref, s_ref, q_ref, mx_ref, mn_ref,
                   deg_ref, w_ref, b_ref, out_ref, delta_sm):
    @pl.when(pl.program_id(0) == 0)
    def _():
        logd_all = jnp.log(deg_full_ref[...] + 1.0)
        delta_sm[0] = jnp.sum(logd_all) / nreal

    delta = delta_sm[0]
    deg = deg_ref[...]
    has = deg > 0.0
    inv = 1.0 / jnp.maximum(deg, 1.0)
    mean = jnp.where(has, s_ref[...] * inv, 0.0)
    var = jnp.maximum(q_ref[...] * inv - mean * mean, 0.0)
    std = jnp.where(has, jnp.sqrt(var + 1e-5), jnp.sqrt(1e-5))
    mx = jnp.where(has, mx_ref[...], 0.0)
    mn = jnp.where(has, mn_ref[...], 0.0)
    agg = jnp.concatenate([mean, mn, mx, std], axis=1)
    logd = jnp.log(deg + 1.0)
    amp = logd / delta
    att = delta / jnp.maximum(logd, 1e-5)
    o = (jnp.dot(agg, w_ref[0:512, :], preferred_element_type=jnp.float32)
         + amp * jnp.dot(agg, w_ref[512:1024, :],
                         preferred_element_type=jnp.float32)
         + att * jnp.dot(agg, w_ref[1024:1536, :],
                         preferred_element_type=jnp.float32)
         + b_ref[...])
    if relu:
        o = jnp.maximum(o, 0.0)
    out_ref[...] = o


def _tc_layer(s, q, mx, mn, deg, w, b, nreal, relu):
    grid = (NPAD // BLK,)
    mom_spec = pl.BlockSpec((BLK, D), lambda i: (i, 0))
    return pl.pallas_call(
        functools.partial(_tc_layer_body, float(nreal), relu),
        grid=grid,
        in_specs=[
            pl.BlockSpec((NPAD, 1), lambda i: (0, 0)),
            mom_spec, mom_spec, mom_spec, mom_spec,
            pl.BlockSpec((BLK, 1), lambda i: (i, 0)),
            pl.BlockSpec((12 * D, D), lambda i: (0, 0)),
            pl.BlockSpec((D,), lambda i: (0,)),
        ],
        out_specs=pl.BlockSpec((BLK, D), lambda i: (i, 0)),
        out_shape=jax.ShapeDtypeStruct((NPAD, D), jnp.float32),
        scratch_shapes=[pltpu.SMEM((1,), jnp.float32)],
    )(deg, s, q, mx, mn, deg, w, b)


def kernel(x, edge_index, edge_weight, W1, b1, W2, b2, W3, b3):
    n = x.shape[0]
    src = edge_index[0]
    dst = edge_index[1]
    dsts, srcs, ews = jax.lax.sort(
        (dst, src, edge_weight), dimension=0, num_keys=1)
    bounds = jnp.arange(NPAD + 16, dtype=jnp.int32)
    nstarts = jnp.searchsorted(dsts, bounds, side="left").astype(jnp.int32)
    srcs = jnp.concatenate([srcs, jnp.zeros((2 * CHUNK,), jnp.int32)])
    ews = jnp.concatenate([ews, jnp.zeros((2 * CHUNK,), jnp.float32)])

    h = jnp.concatenate(
        [x, jnp.zeros((NPAD - n, D), jnp.float32)], axis=0)
    for w, b, relu in ((W1, b1, True), (W2, b2, True), (W3, b3, False)):
        s, q, mx, mn, deg = _sc_moments(h, srcs, ews, nstarts)
        h = _tc_layer(s.reshape(NPAD, D), q.reshape(NPAD, D),
                      mx.reshape(NPAD, D), mn.reshape(NPAD, D),
                      deg.reshape(NPAD, 16)[:, :1], w, b, n, relu)
    return h[:n]


# speedup vs baseline: 1.7227x; 1.7227x over previous
import functools

import jax
import jax.numpy as jnp
from jax import lax
from jax.experimental import pallas as pl
from jax.experimental.pallas import tpu as pltpu
from jax.experimental.pallas import tpu_sc as plsc

N_EDGES = 320000
D = 128
NV = D // 16
NC, NS = 2, 16
NW = NC * NS
R = 2
NPR = 160
NRANGE = NW * R
NPAD = NRANGE * NPR
CHUNK = 128
EPW = N_EDGES // NW
SCHUNK = 80
EDPAD = N_EDGES + 2 * CHUNK
FMAX = 3.4e38

_MESH = plsc.VectorSubcoreMesh(
    core_axis_name="c", subcore_axis_name="s",
    num_cores=NC, num_subcores=NS)
_NO_LAYOUT = pltpu.CompilerParams(needs_layout_passes=False)


def _wid():
    return lax.axis_index("c") * NS + lax.axis_index("s")


def _sc_hist(dst):
    @functools.partial(
        pl.kernel, mesh=_MESH, compiler_params=_NO_LAYOUT,
        out_type=jax.ShapeDtypeStruct((NW * NPAD,), jnp.float32),
        scratch_types=[pltpu.VMEM((EPW,), jnp.int32),
                       pltpu.VMEM((NPAD,), jnp.float32)])
    def body(dst_hbm, o_hist, v_dst, v_hist):
        wid = _wid()
        pltpu.sync_copy(dst_hbm.at[pl.ds(wid * EPW, EPW)], v_dst)
        zi = jnp.zeros((16,), jnp.float32)
        for i in range(NPAD // 16):
            v_hist[pl.ds(i * 16, 16)] = zi
        ones = jnp.ones((16,), jnp.float32)

        def g_body(g, carry):
            dvec = v_dst[pl.ds(g * 16, 16)]
            plsc.addupdate_scatter(v_hist, [dvec], ones)
            return carry

        lax.fori_loop(0, EPW // 16, g_body, 0)
        pltpu.sync_copy(v_hist, o_hist.at[pl.ds(wid * NPAD, NPAD)])

    return body(dst)


def _sc_scatter(dst, src, ew_i, off):
    scratch_types = [
        pltpu.VMEM((EPW,), jnp.int32),
        pltpu.VMEM((EPW,), jnp.int32),
        pltpu.VMEM((EPW,), jnp.int32),
        pltpu.VMEM((NPAD,), jnp.float32),
        pltpu.VMEM((SCHUNK, 128), jnp.int32),
        pltpu.VMEM((SCHUNK, 128), jnp.int32),
        pltpu.VMEM((SCHUNK,), jnp.int32),
        pltpu.VMEM((SCHUNK,), jnp.int32),
        pltpu.SemaphoreType.DMA,
        pltpu.SemaphoreType.DMA,
    ]

    @functools.partial(
        pl.kernel, mesh=_MESH, compiler_params=_NO_LAYOUT,
        out_type=jax.ShapeDtypeStruct((EDPAD, 128), jnp.int32),
        scratch_types=scratch_types)
    def body(dst_hbm, src_hbm, ew_hbm, off_hbm, o_ed,
             v_dst, v_src, v_ew, v_off, st0, st1, po0, po1, sem0, sem1):
        wid = _wid()
        eb = wid * EPW
        pltpu.sync_copy(dst_hbm.at[pl.ds(eb, EPW)], v_dst)
        pltpu.sync_copy(src_hbm.at[pl.ds(eb, EPW)], v_src)
        pltpu.sync_copy(ew_hbm.at[pl.ds(eb, EPW)], v_ew)
        pltpu.sync_copy(off_hbm.at[pl.ds(wid * NPAD, NPAD)], v_off)
        lane = lax.iota(jnp.int32, 16)
        ones = jnp.ones((16,), jnp.float32)
        zi = jnp.zeros((16,), jnp.int32)

        def do_chunk(c, stage, pos, sem):
            for g in range(SCHUNK // 16):
                lb = c * SCHUNK + g * 16
                dvec = v_dst[pl.ds(lb, 16)]
                svec = v_src[pl.ds(lb, 16)]
                wvec = v_ew[pl.ds(lb, 16)]
                stale = plsc.load_gather(v_off, [dvec])
                rank = zi
                for s in range(1, 16):
                    idxc = jnp.maximum(lane - s, 0)
                    sh = dvec.at[idxc].get(mode="promise_in_bounds")
                    rank = rank + jnp.where(
                        (lane >= s) & (sh == dvec), 1, 0).astype(jnp.int32)
                plsc.addupdate_scatter(v_off, [dvec], ones)
                rowv = g * 16 + lane
                plsc.store_scatter(stage, [rowv, zi], svec)
                plsc.store_scatter(stage, [rowv, zi + 1], wvec)
                pos[pl.ds(g * 16, 16)] = (
                    stale + rank.astype(jnp.float32)).astype(jnp.int32)
            pltpu.async_copy(stage, o_ed.at[pos], sem)

        def wait_chunk(stage, pos, sem):
            pltpu.make_async_copy(stage, o_ed.at[pos], sem).wait()

        def chunk_loop(c, carry):
            @pl.when(jnp.bitwise_and(c, 1) == 0)
            def _():
                @pl.when(c >= 2)
                def _():
                    wait_chunk(st0, po0, sem0)
                do_chunk(c, st0, po0, sem0)

            @pl.when(jnp.bitwise_and(c, 1) == 1)
            def _():
                @pl.when(c >= 2)
                def _():
                    wait_chunk(st1, po1, sem1)
                do_chunk(c, st1, po1, sem1)

            return carry

        lax.fori_loop(0, EPW // SCHUNK, chunk_loop, 0)
        wait_chunk(st0, po0, sem0)
        wait_chunk(st1, po1, sem1)

    return body(dst, src, ew_i, off)


def _sc_compact(edata):
    out_type = [
        jax.ShapeDtypeStruct((N_EDGES + 2 * CHUNK,), jnp.int32),
        jax.ShapeDtypeStruct((N_EDGES + 2 * CHUNK,), jnp.int32),
    ]
    scratch_types = [
        pltpu.VMEM((2 * SCHUNK, 128), jnp.int32),
        pltpu.VMEM((SCHUNK,), jnp.int32),
        pltpu.VMEM((SCHUNK,), jnp.int32),
        pltpu.SemaphoreType.DMA,
        pltpu.SemaphoreType.DMA,
    ]

    @functools.partial(
        pl.kernel, mesh=_MESH, compiler_params=_NO_LAYOUT,
        out_type=out_type, scratch_types=scratch_types)
    def body(ed_hbm, o_src, o_ew, v_buf, v_so, v_wo, sem0, sem1):
        wid = _wid()
        eb = wid * EPW
        sems = (sem0, sem1)
        lane = lax.iota(jnp.int32, 16)
        zi = jnp.zeros((16,), jnp.int32)
        nch = EPW // SCHUNK

        def issue(c, h):
            pltpu.async_copy(ed_hbm.at[pl.ds(eb + c * SCHUNK, SCHUNK)],
                             v_buf.at[pl.ds(h * SCHUNK, SCHUNK)], sems[h])

        def wait(c, h):
            pltpu.make_async_copy(
                ed_hbm.at[pl.ds(eb + c * SCHUNK, SCHUNK)],
                v_buf.at[pl.ds(h * SCHUNK, SCHUNK)], sems[h]).wait()

        def process(c, h):
            wait(c, h)
            for g in range(SCHUNK // 16):
                rvec = (h * SCHUNK + g * 16) + lane
                sv = plsc.load_gather(v_buf, [rvec, zi])
                wv = plsc.load_gather(v_buf, [rvec, zi + 1])
                v_so[pl.ds(g * 16, 16)] = sv
                v_wo[pl.ds(g * 16, 16)] = wv
            ob = eb + c * SCHUNK
            pltpu.sync_copy(v_so, o_src.at[pl.ds(ob, SCHUNK)])
            pltpu.sync_copy(v_wo, o_ew.at[pl.ds(ob, SCHUNK)])

        issue(0, 0)
        issue(1, 1)

        def loop(c, carry):
            @pl.when(jnp.bitwise_and(c, 1) == 0)
            def _():
                process(c, 0)

                @pl.when(c + 2 < nch)
                def _():
                    issue(c + 2, 0)

            @pl.when(jnp.bitwise_and(c, 1) == 1)
            def _():
                process(c, 1)

                @pl.when(c + 2 < nch)
                def _():
                    issue(c + 2, 1)

            return carry

        lax.fori_loop(0, nch, loop, 0)

    return body(edata)


def _sc_moments(x_pad, srcs, ews, nstarts):
    out_type = [
        jax.ShapeDtypeStruct((NPAD * D,), jnp.float32),
        jax.ShapeDtypeStruct((NPAD * D,), jnp.float32),
        jax.ShapeDtypeStruct((NPAD * D,), jnp.float32),
        jax.ShapeDtypeStruct((NPAD * D,), jnp.float32),
        jax.ShapeDtypeStruct((NPAD * 16,), jnp.float32),
    ]
    scratch_types = [
        pltpu.VMEM((NPR * D,), jnp.float32),
        pltpu.VMEM((NPR * D,), jnp.float32),
        pltpu.VMEM((NPR * D,), jnp.float32),
        pltpu.VMEM((NPR * D,), jnp.float32),
        pltpu.VMEM((NPR * 16,), jnp.float32),
        pltpu.VMEM((2 * CHUNK, D), jnp.float32),
        pltpu.VMEM((2 * CHUNK,), jnp.int32),
        pltpu.VMEM((2 * CHUNK + 16,), jnp.float32),
        pltpu.VMEM((NPR + 16,), jnp.int32),
        pltpu.SemaphoreType.DMA,
        pltpu.SemaphoreType.DMA,
    ]

    @functools.partial(pl.kernel, out_type=out_type, mesh=_MESH,
                       scratch_types=scratch_types)
    def body(x_hbm, srcs_hbm, ews_hbm, nstarts_hbm,
             o_sum, o_sq, o_mx, o_mn, o_deg,
             v_sum, v_sq, v_mx, v_mn, v_deg,
             v_rows, v_idx, v_ew, v_nst, sem0, sem1):
        wid = _wid()
        zeros = jnp.zeros((16,), jnp.float32)
        fmaxv = jnp.full((16,), FMAX, jnp.float32)
        sems = (sem0, sem1)

        def stage_issue(ck, h):
            cb = ck * CHUNK
            hb = h * CHUNK
            pltpu.sync_copy(srcs_hbm.at[pl.ds(cb, CHUNK)],
                            v_idx.at[pl.ds(hb, CHUNK)])
            pltpu.sync_copy(ews_hbm.at[pl.ds(cb, CHUNK)],
                            v_ew.at[pl.ds(hb, CHUNK)])
            for g in range(CHUNK // 16):
                sl = pl.ds(hb + g * 16, 16)
                vals = v_idx[sl]
                v_idx[sl] = jnp.minimum(jnp.maximum(vals, 0), NPAD - 1)
            pltpu.async_copy(x_hbm.at[v_idx.at[pl.ds(hb, CHUNK)]],
                             v_rows.at[pl.ds(hb, CHUNK)], sems[h])

        def wait_gather(h):
            hb = h * CHUNK
            pltpu.make_async_copy(
                x_hbm.at[v_idx.at[pl.ds(hb, CHUNK)]],
                v_rows.at[pl.ds(hb, CHUNK)], sems[h]).wait()

        def refill(c):
            c1 = c + 1
            odd = jnp.bitwise_and(c1, 1)

            @pl.when(odd == 0)
            def _():
                wait_gather(0)
                stage_issue(c1 + 1, 1)

            @pl.when(odd == 1)
            def _():
                wait_gather(1)
                stage_issue(c1 + 1, 0)

            return c1

        def edge_body(j, st):
            (c, eptr), (s_acc, q_acc, x_acc, n_acc) = st
            c, eptr = lax.cond(
                eptr == CHUNK,
                lambda: (refill(c), jnp.int32(0)),
                lambda: (c, eptr))
            off = jnp.bitwise_and(c, 1) * CHUNK + eptr
            w = v_ew[pl.ds(off, 16)][0]
            s_new, q_new, x_new, n_new = [], [], [], []
            for f in range(NV):
                row = v_rows[off, pl.ds(f * 16, 16)]
                m = row * w
                s_new.append(s_acc[f] + m)
                q_new.append(q_acc[f] + m * m)
                x_new.append(jnp.maximum(x_acc[f], m))
                n_new.append(jnp.minimum(n_acc[f], m))
            return ((c, eptr + 1),
                    (tuple(s_new), tuple(q_new),
                     tuple(x_new), tuple(n_new)))

        acc_fresh = (tuple(zeros for _ in range(NV)),
                     tuple(zeros for _ in range(NV)),
                     tuple(-fmaxv for _ in range(NV)),
                     tuple(fmaxv for _ in range(NV)))

        def node_body(i, st):
            nv = v_nst[pl.ds(i, 16)]
            ne = nv[1] - nv[0]
            st2 = lax.fori_loop(0, ne, edge_body, (st, acc_fresh))
            (c, eptr), (s_acc, q_acc, x_acc, n_acc) = st2
            lo = i * D
            for f in range(NV):
                sl = pl.ds(lo + f * 16, 16)
                v_sum[sl] = s_acc[f]
                v_sq[sl] = q_acc[f]
                v_mx[sl] = x_acc[f]
                v_mn[sl] = n_acc[f]
            v_deg[pl.ds(i * 16, 16)] = zeros + ne.astype(jnp.float32)
            return (c, eptr)

        for r in range(R):
            rid = wid * R + r
            base = rid * NPR
            pltpu.sync_copy(nstarts_hbm.at[pl.ds(base, NPR + 16)], v_nst)
            start = v_nst[pl.ds(0, 16)][0]
            c0 = start // CHUNK
            odd0 = jnp.bitwise_and(c0, 1)

            @pl.when(odd0 == 0)
            def _():
                stage_issue(c0, 0)
                stage_issue(c0 + 1, 1)
                wait_gather(0)

            @pl.when(odd0 == 1)
            def _():
                stage_issue(c0, 1)
                stage_issue(c0 + 1, 0)
                wait_gather(1)

            c, eptr = lax.fori_loop(
                0, NPR, node_body, (c0, start - c0 * CHUNK))

            odd1 = jnp.bitwise_and(c + 1, 1)

            @pl.when(odd1 == 0)
            def _():
                wait_gather(0)

            @pl.when(odd1 == 1)
            def _():
                wait_gather(1)

            pltpu.sync_copy(v_sum, o_sum.at[pl.ds(base * D, NPR * D)])
            pltpu.sync_copy(v_sq, o_sq.at[pl.ds(base * D, NPR * D)])
            pltpu.sync_copy(v_mx, o_mx.at[pl.ds(base * D, NPR * D)])
            pltpu.sync_copy(v_mn, o_mn.at[pl.ds(base * D, NPR * D)])
            pltpu.sync_copy(v_deg, o_deg.at[pl.ds(base * 16, NPR * 16)])

    return body(x_pad, srcs, ews, nstarts)


BLK = 1024


def _tc_layer_body(nreal, relu, deg_full_ref, s_ref, q_ref, mx_ref, mn_ref,
                   deg_ref, w_ref, b_ref, out_ref, delta_sm):
    @pl.when(pl.program_id(0) == 0)
    def _():
        logd_all = jnp.log(deg_full_ref[...] + 1.0)
        delta_sm[0] = jnp.sum(logd_all) / nreal

    delta = delta_sm[0]
    deg = deg_ref[...]
    has = deg > 0.0
    inv = 1.0 / jnp.maximum(deg, 1.0)
    mean = jnp.where(has, s_ref[...] * inv, 0.0)
    var = jnp.maximum(q_ref[...] * inv - mean * mean, 0.0)
    std = jnp.where(has, jnp.sqrt(var + 1e-5), jnp.sqrt(1e-5))
    mx = jnp.where(has, mx_ref[...], 0.0)
    mn = jnp.where(has, mn_ref[...], 0.0)
    agg = jnp.concatenate([mean, mn, mx, std], axis=1)
    logd = jnp.log(deg + 1.0)
    amp = logd / delta
    att = delta / jnp.maximum(logd, 1e-5)
    o = (jnp.dot(agg, w_ref[0:512, :], preferred_element_type=jnp.float32)
         + amp * jnp.dot(agg, w_ref[512:1024, :],
                         preferred_element_type=jnp.float32)
         + att * jnp.dot(agg, w_ref[1024:1536, :],
                         preferred_element_type=jnp.float32)
         + b_ref[...])
    if relu:
        o = jnp.maximum(o, 0.0)
    out_ref[...] = o


def _tc_layer(s, q, mx, mn, deg, w, b, nreal, relu):
    grid = (NPAD // BLK,)
    mom_spec = pl.BlockSpec((BLK, D), lambda i: (i, 0))
    return pl.pallas_call(
        functools.partial(_tc_layer_body, float(nreal), relu),
        grid=grid,
        in_specs=[
            pl.BlockSpec((NPAD, 1), lambda i: (0, 0)),
            mom_spec, mom_spec, mom_spec, mom_spec,
            pl.BlockSpec((BLK, 1), lambda i: (i, 0)),
            pl.BlockSpec((12 * D, D), lambda i: (0, 0)),
            pl.BlockSpec((D,), lambda i: (0,)),
        ],
        out_specs=pl.BlockSpec((BLK, D), lambda i: (i, 0)),
        out_shape=jax.ShapeDtypeStruct((NPAD, D), jnp.float32),
        scratch_shapes=[pltpu.SMEM((1,), jnp.float32)],
    )(deg, s, q, mx, mn, deg, w, b)


def kernel(x, edge_index, edge_weight, W1, b1, W2, b2, W3, b3):
    n = x.shape[0]
    src = edge_index[0]
    dst = edge_index[1]
    hist = _sc_hist(dst).reshape(NW, NPAD).astype(jnp.int32)
    csum_w = jnp.cumsum(hist, axis=0)
    tot = csum_w[-1]
    gbase = (jnp.cumsum(tot) - tot).astype(jnp.int32)
    off = (gbase[None, :] + (csum_w - hist)).astype(jnp.float32)
    ew_i = lax.bitcast_convert_type(edge_weight, jnp.int32)
    edata = _sc_scatter(dst, src, ew_i, off.reshape(NW * NPAD))
    srcs, ews_bits = _sc_compact(edata)
    ews = lax.bitcast_convert_type(ews_bits, jnp.float32)
    nstarts = jnp.concatenate(
        [gbase, jnp.full((16,), N_EDGES, jnp.int32)])

    h = jnp.concatenate(
        [x, jnp.zeros((NPAD - n, D), jnp.float32)], axis=0)
    for w, b, relu in ((W1, b1, True), (W2, b2, True), (W3, b3, False)):
        s, q, mx, mn, deg = _sc_moments(h, srcs, ews, nstarts)
        h = _tc_layer(s.reshape(NPAD, D), q.reshape(NPAD, D),
                      mx.reshape(NPAD, D), mn.reshape(NPAD, D),
                      deg.reshape(NPAD, 16)[:, :1], w, b, n, relu)
    return h[:n]


# speedup vs baseline: 2.2577x; 1.3105x over previous
import functools

import jax
import jax.numpy as jnp
from jax import lax
from jax.experimental import pallas as pl
from jax.experimental.pallas import tpu as pltpu
from jax.experimental.pallas import tpu_sc as plsc

N_EDGES = 320000
D = 128
NV = D // 16
NC, NS = 2, 16
NW = NC * NS
R = 2
NPR = 160
NRANGE = NW * R
NPAD = NRANGE * NPR
CHUNK = 128
EPW = N_EDGES // NW
SCHUNK = 80
EDPAD = N_EDGES + 2 * CHUNK
FMAX = 3.4e38

_MESH = plsc.VectorSubcoreMesh(
    core_axis_name="c", subcore_axis_name="s",
    num_cores=NC, num_subcores=NS)
_NO_LAYOUT = pltpu.CompilerParams(needs_layout_passes=False)


def _wid():
    return lax.axis_index("c") * NS + lax.axis_index("s")


def _sc_hist(dst):
    @functools.partial(
        pl.kernel, mesh=_MESH, compiler_params=_NO_LAYOUT,
        out_type=jax.ShapeDtypeStruct((NW * NPAD,), jnp.float32),
        scratch_types=[pltpu.VMEM((EPW,), jnp.int32),
                       pltpu.VMEM((NPAD,), jnp.float32)])
    def body(dst_hbm, o_hist, v_dst, v_hist):
        wid = _wid()
        pltpu.sync_copy(dst_hbm.at[pl.ds(wid * EPW, EPW)], v_dst)
        zi = jnp.zeros((16,), jnp.float32)
        for i in range(NPAD // 16):
            v_hist[pl.ds(i * 16, 16)] = zi
        ones = jnp.ones((16,), jnp.float32)

        def g_body(g, carry):
            dvec = v_dst[pl.ds(g * 16, 16)]
            plsc.addupdate_scatter(v_hist, [dvec], ones)
            return carry

        lax.fori_loop(0, EPW // 16, g_body, 0)
        pltpu.sync_copy(v_hist, o_hist.at[pl.ds(wid * NPAD, NPAD)])

    return body(dst)


def _sc_scatter(dst, src, ew_i, off):
    scratch_types = [
        pltpu.VMEM((EPW,), jnp.int32),
        pltpu.VMEM((EPW,), jnp.int32),
        pltpu.VMEM((EPW,), jnp.int32),
        pltpu.VMEM((NPAD,), jnp.float32),
        pltpu.VMEM((SCHUNK, 128), jnp.int32),
        pltpu.VMEM((SCHUNK, 128), jnp.int32),
        pltpu.VMEM((SCHUNK,), jnp.int32),
        pltpu.VMEM((SCHUNK,), jnp.int32),
        pltpu.SemaphoreType.DMA,
        pltpu.SemaphoreType.DMA,
    ]

    @functools.partial(
        pl.kernel, mesh=_MESH, compiler_params=_NO_LAYOUT,
        out_type=jax.ShapeDtypeStruct((EDPAD, 128), jnp.int32),
        scratch_types=scratch_types)
    def body(dst_hbm, src_hbm, ew_hbm, off_hbm, o_ed,
             v_dst, v_src, v_ew, v_off, st0, st1, po0, po1, sem0, sem1):
        wid = _wid()
        eb = wid * EPW
        pltpu.sync_copy(dst_hbm.at[pl.ds(eb, EPW)], v_dst)
        pltpu.sync_copy(src_hbm.at[pl.ds(eb, EPW)], v_src)
        pltpu.sync_copy(ew_hbm.at[pl.ds(eb, EPW)], v_ew)
        pltpu.sync_copy(off_hbm.at[pl.ds(wid * NPAD, NPAD)], v_off)
        lane = lax.iota(jnp.int32, 16)
        ones = jnp.ones((16,), jnp.float32)
        zi = jnp.zeros((16,), jnp.int32)

        def do_chunk(c, stage, pos, sem):
            for g in range(SCHUNK // 16):
                lb = c * SCHUNK + g * 16
                dvec = v_dst[pl.ds(lb, 16)]
                svec = v_src[pl.ds(lb, 16)]
                wvec = v_ew[pl.ds(lb, 16)]
                stale = plsc.load_gather(v_off, [dvec])
                rank = zi
                for s in range(1, 16):
                    idxc = jnp.maximum(lane - s, 0)
                    sh = dvec.at[idxc].get(mode="promise_in_bounds")
                    rank = rank + jnp.where(
                        (lane >= s) & (sh == dvec), 1, 0).astype(jnp.int32)
                plsc.addupdate_scatter(v_off, [dvec], ones)
                rowv = g * 16 + lane
                plsc.store_scatter(stage, [rowv, zi], svec)
                plsc.store_scatter(stage, [rowv, zi + 1], wvec)
                pos[pl.ds(g * 16, 16)] = (
                    stale + rank.astype(jnp.float32)).astype(jnp.int32)
            pltpu.async_copy(stage, o_ed.at[pos], sem)

        def wait_chunk(stage, pos, sem):
            pltpu.make_async_copy(stage, o_ed.at[pos], sem).wait()

        def chunk_loop(c, carry):
            @pl.when(jnp.bitwise_and(c, 1) == 0)
            def _():
                @pl.when(c >= 2)
                def _():
                    wait_chunk(st0, po0, sem0)
                do_chunk(c, st0, po0, sem0)

            @pl.when(jnp.bitwise_and(c, 1) == 1)
            def _():
                @pl.when(c >= 2)
                def _():
                    wait_chunk(st1, po1, sem1)
                do_chunk(c, st1, po1, sem1)

            return carry

        lax.fori_loop(0, EPW // SCHUNK, chunk_loop, 0)
        wait_chunk(st0, po0, sem0)
        wait_chunk(st1, po1, sem1)

    return body(dst, src, ew_i, off)


def _sc_compact(edata):
    out_type = [
        jax.ShapeDtypeStruct((N_EDGES + 2 * CHUNK,), jnp.int32),
        jax.ShapeDtypeStruct((N_EDGES + 2 * CHUNK,), jnp.int32),
    ]
    scratch_types = [
        pltpu.VMEM((2 * SCHUNK, 128), jnp.int32),
        pltpu.VMEM((SCHUNK,), jnp.int32),
        pltpu.VMEM((SCHUNK,), jnp.int32),
        pltpu.SemaphoreType.DMA,
        pltpu.SemaphoreType.DMA,
    ]

    @functools.partial(
        pl.kernel, mesh=_MESH, compiler_params=_NO_LAYOUT,
        out_type=out_type, scratch_types=scratch_types)
    def body(ed_hbm, o_src, o_ew, v_buf, v_so, v_wo, sem0, sem1):
        wid = _wid()
        eb = wid * EPW
        sems = (sem0, sem1)
        lane = lax.iota(jnp.int32, 16)
        zi = jnp.zeros((16,), jnp.int32)
        nch = EPW // SCHUNK

        def issue(c, h):
            pltpu.async_copy(ed_hbm.at[pl.ds(eb + c * SCHUNK, SCHUNK)],
                             v_buf.at[pl.ds(h * SCHUNK, SCHUNK)], sems[h])

        def wait(c, h):
            pltpu.make_async_copy(
                ed_hbm.at[pl.ds(eb + c * SCHUNK, SCHUNK)],
                v_buf.at[pl.ds(h * SCHUNK, SCHUNK)], sems[h]).wait()

        def process(c, h):
            wait(c, h)
            for g in range(SCHUNK // 16):
                rvec = (h * SCHUNK + g * 16) + lane
                sv = plsc.load_gather(v_buf, [rvec, zi])
                wv = plsc.load_gather(v_buf, [rvec, zi + 1])
                v_so[pl.ds(g * 16, 16)] = sv
                v_wo[pl.ds(g * 16, 16)] = wv
            ob = eb + c * SCHUNK
            pltpu.sync_copy(v_so, o_src.at[pl.ds(ob, SCHUNK)])
            pltpu.sync_copy(v_wo, o_ew.at[pl.ds(ob, SCHUNK)])

        issue(0, 0)
        issue(1, 1)

        def loop(c, carry):
            @pl.when(jnp.bitwise_and(c, 1) == 0)
            def _():
                process(c, 0)

                @pl.when(c + 2 < nch)
                def _():
                    issue(c + 2, 0)

            @pl.when(jnp.bitwise_and(c, 1) == 1)
            def _():
                process(c, 1)

                @pl.when(c + 2 < nch)
                def _():
                    issue(c + 2, 1)

            return carry

        lax.fori_loop(0, nch, loop, 0)

    return body(edata)


def _sc_moments(x_pad, srcs, ews, nstarts):
    out_type = [
        jax.ShapeDtypeStruct((NPAD * D,), jnp.float32),
        jax.ShapeDtypeStruct((NPAD * D,), jnp.float32),
        jax.ShapeDtypeStruct((NPAD * D,), jnp.float32),
        jax.ShapeDtypeStruct((NPAD * D,), jnp.float32),
        jax.ShapeDtypeStruct((NPAD * 16,), jnp.float32),
    ]
    scratch_types = [
        pltpu.VMEM((NPR * D,), jnp.float32),
        pltpu.VMEM((NPR * D,), jnp.float32),
        pltpu.VMEM((NPR * D,), jnp.float32),
        pltpu.VMEM((NPR * D,), jnp.float32),
        pltpu.VMEM((NPR * 16,), jnp.float32),
        pltpu.VMEM((2 * CHUNK, D), jnp.float32),
        pltpu.VMEM((2 * CHUNK,), jnp.int32),
        pltpu.VMEM((2 * CHUNK + 16,), jnp.float32),
        pltpu.VMEM((NPR + 16,), jnp.int32),
        pltpu.SemaphoreType.DMA,
        pltpu.SemaphoreType.DMA,
    ]

    @functools.partial(pl.kernel, out_type=out_type, mesh=_MESH,
                       compiler_params=_NO_LAYOUT,
                       scratch_types=scratch_types)
    def body(x_hbm, srcs_hbm, ews_hbm, nstarts_hbm,
             o_sum, o_sq, o_mx, o_mn, o_deg,
             v_sum, v_sq, v_mx, v_mn, v_deg,
             v_rows, v_idx, v_ew, v_nst, sem0, sem1):
        wid = _wid()
        zeros = jnp.zeros((16,), jnp.float32)
        fmaxv = jnp.full((16,), FMAX, jnp.float32)
        sems = (sem0, sem1)

        def stage_issue(ck, h):
            cb = ck * CHUNK
            hb = h * CHUNK
            pltpu.sync_copy(srcs_hbm.at[pl.ds(cb, CHUNK)],
                            v_idx.at[pl.ds(hb, CHUNK)])
            pltpu.sync_copy(ews_hbm.at[pl.ds(cb, CHUNK)],
                            v_ew.at[pl.ds(hb, CHUNK)])
            for g in range(CHUNK // 16):
                sl = pl.ds(hb + g * 16, 16)
                vals = v_idx[sl]
                v_idx[sl] = jnp.minimum(jnp.maximum(vals, 0), NPAD - 1)
            pltpu.async_copy(x_hbm.at[v_idx.at[pl.ds(hb, CHUNK)]],
                             v_rows.at[pl.ds(hb, CHUNK)], sems[h])

        def wait_gather(h):
            hb = h * CHUNK
            pltpu.make_async_copy(
                x_hbm.at[v_idx.at[pl.ds(hb, CHUNK)]],
                v_rows.at[pl.ds(hb, CHUNK)], sems[h]).wait()

        def refill(c):
            c1 = c + 1
            odd = jnp.bitwise_and(c1, 1)

            @pl.when(odd == 0)
            def _():
                wait_gather(0)
                stage_issue(c1 + 1, 1)

            @pl.when(odd == 1)
            def _():
                wait_gather(1)
                stage_issue(c1 + 1, 0)

            return c1

        acc_fresh = (tuple(zeros for _ in range(NV)),
                     tuple(zeros for _ in range(NV)),
                     tuple(-fmaxv for _ in range(NV)),
                     tuple(fmaxv for _ in range(NV)))

        def run_loop(s):
            c, eptr, rem, accs = s
            c, eptr = lax.cond(
                eptr == CHUNK,
                lambda: (refill(c), jnp.int32(0)),
                lambda: (c, eptr))
            boff = jnp.bitwise_and(c, 1) * CHUNK + eptr
            run = jnp.minimum(rem, CHUNK - eptr)

            def e_body(j, accs2):
                s_acc, q_acc, x_acc, n_acc = accs2
                off = boff + j
                w = v_ew[pl.ds(off, 16)][0]
                s_new, q_new, x_new, n_new = [], [], [], []
                for f in range(NV):
                    row = v_rows[off, pl.ds(f * 16, 16)]
                    m = row * w
                    s_new.append(s_acc[f] + m)
                    q_new.append(q_acc[f] + m * m)
                    x_new.append(jnp.maximum(x_acc[f], m))
                    n_new.append(jnp.minimum(n_acc[f], m))
                return (tuple(s_new), tuple(q_new),
                        tuple(x_new), tuple(n_new))

            accs = lax.fori_loop(0, run, e_body, accs)
            return (c, eptr + run, rem - run, accs)

        def node_body(i, st):
            c, eptr = st
            nv = v_nst[pl.ds(i, 16)]
            ne = nv[1] - nv[0]
            c, eptr, _, (s_acc, q_acc, x_acc, n_acc) = lax.while_loop(
                lambda s: s[2] > 0, run_loop,
                (c, eptr, ne, acc_fresh))
            lo = i * D
            for f in range(NV):
                sl = pl.ds(lo + f * 16, 16)
                v_sum[sl] = s_acc[f]
                v_sq[sl] = q_acc[f]
                v_mx[sl] = x_acc[f]
                v_mn[sl] = n_acc[f]
            v_deg[pl.ds(i * 16, 16)] = zeros + ne.astype(jnp.float32)
            return (c, eptr)

        for r in range(R):
            rid = wid * R + r
            base = rid * NPR
            pltpu.sync_copy(nstarts_hbm.at[pl.ds(base, NPR + 16)], v_nst)
            start = v_nst[pl.ds(0, 16)][0]
            c0 = start // CHUNK
            odd0 = jnp.bitwise_and(c0, 1)

            @pl.when(odd0 == 0)
            def _():
                stage_issue(c0, 0)
                stage_issue(c0 + 1, 1)
                wait_gather(0)

            @pl.when(odd0 == 1)
            def _():
                stage_issue(c0, 1)
                stage_issue(c0 + 1, 0)
                wait_gather(1)

            c, eptr = lax.fori_loop(
                0, NPR, node_body, (c0, start - c0 * CHUNK))

            odd1 = jnp.bitwise_and(c + 1, 1)

            @pl.when(odd1 == 0)
            def _():
                wait_gather(0)

            @pl.when(odd1 == 1)
            def _():
                wait_gather(1)

            pltpu.sync_copy(v_sum, o_sum.at[pl.ds(base * D, NPR * D)])
            pltpu.sync_copy(v_sq, o_sq.at[pl.ds(base * D, NPR * D)])
            pltpu.sync_copy(v_mx, o_mx.at[pl.ds(base * D, NPR * D)])
            pltpu.sync_copy(v_mn, o_mn.at[pl.ds(base * D, NPR * D)])
            pltpu.sync_copy(v_deg, o_deg.at[pl.ds(base * 16, NPR * 16)])

    return body(x_pad, srcs, ews, nstarts)


BLK = 1024


def _tc_layer_body(nreal, relu, deg_full_ref, s_ref, q_ref, mx_ref, mn_ref,
                   deg_ref, w_ref, b_ref, out_ref, delta_sm):
    @pl.when(pl.program_id(0) == 0)
    def _():
        logd_all = jnp.log(deg_full_ref[...] + 1.0)
        delta_sm[0] = jnp.sum(logd_all) / nreal

    delta = delta_sm[0]
    deg = deg_ref[...]
    has = deg > 0.0
    inv = 1.0 / jnp.maximum(deg, 1.0)
    mean = jnp.where(has, s_ref[...] * inv, 0.0)
    var = jnp.maximum(q_ref[...] * inv - mean * mean, 0.0)
    std = jnp.where(has, jnp.sqrt(var + 1e-5), jnp.sqrt(1e-5))
    mx = jnp.where(has, mx_ref[...], 0.0)
    mn = jnp.where(has, mn_ref[...], 0.0)
    agg = jnp.concatenate([mean, mn, mx, std], axis=1)
    logd = jnp.log(deg + 1.0)
    amp = logd / delta
    att = delta / jnp.maximum(logd, 1e-5)
    o = (jnp.dot(agg, w_ref[0:512, :], preferred_element_type=jnp.float32)
         + amp * jnp.dot(agg, w_ref[512:1024, :],
                         preferred_element_type=jnp.float32)
         + att * jnp.dot(agg, w_ref[1024:1536, :],
                         preferred_element_type=jnp.float32)
         + b_ref[...])
    if relu:
        o = jnp.maximum(o, 0.0)
    out_ref[...] = o


def _tc_layer(s, q, mx, mn, deg, w, b, nreal, relu):
    grid = (NPAD // BLK,)
    mom_spec = pl.BlockSpec((BLK, D), lambda i: (i, 0))
    return pl.pallas_call(
        functools.partial(_tc_layer_body, float(nreal), relu),
        grid=grid,
        in_specs=[
            pl.BlockSpec((NPAD, 1), lambda i: (0, 0)),
            mom_spec, mom_spec, mom_spec, mom_spec,
            pl.BlockSpec((BLK, 1), lambda i: (i, 0)),
            pl.BlockSpec((12 * D, D), lambda i: (0, 0)),
            pl.BlockSpec((D,), lambda i: (0,)),
        ],
        out_specs=pl.BlockSpec((BLK, D), lambda i: (i, 0)),
        out_shape=jax.ShapeDtypeStruct((NPAD, D), jnp.float32),
        scratch_shapes=[pltpu.SMEM((1,), jnp.float32)],
    )(deg, s, q, mx, mn, deg, w, b)


def kernel(x, edge_index, edge_weight, W1, b1, W2, b2, W3, b3):
    n = x.shape[0]
    src = edge_index[0]
    dst = edge_index[1]
    hist = _sc_hist(dst).reshape(NW, NPAD).astype(jnp.int32)
    csum_w = jnp.cumsum(hist, axis=0)
    tot = csum_w[-1]
    gbase = (jnp.cumsum(tot) - tot).astype(jnp.int32)
    off = (gbase[None, :] + (csum_w - hist)).astype(jnp.float32)
    ew_i = lax.bitcast_convert_type(edge_weight, jnp.int32)
    edata = _sc_scatter(dst, src, ew_i, off.reshape(NW * NPAD))
    srcs, ews_bits = _sc_compact(edata)
    ews = lax.bitcast_convert_type(ews_bits, jnp.float32)
    nstarts = jnp.concatenate(
        [gbase, jnp.full((16,), N_EDGES, jnp.int32)])

    h = jnp.concatenate(
        [x, jnp.zeros((NPAD - n, D), jnp.float32)], axis=0)
    for w, b, relu in ((W1, b1, True), (W2, b2, True), (W3, b3, False)):
        s, q, mx, mn, deg = _sc_moments(h, srcs, ews, nstarts)
        h = _tc_layer(s.reshape(NPAD, D), q.reshape(NPAD, D),
                      mx.reshape(NPAD, D), mn.reshape(NPAD, D),
                      deg.reshape(NPAD, 16)[:, :1], w, b, n, relu)
    return h[:n]
